# baseline (speedup 1.0000x reference)
import jax
import jax.numpy as jnp
from jax import lax
from jax.experimental import pallas as pl
from jax.experimental.pallas import tpu as pltpu
from jax.experimental.pallas import tpu_sc as plsc

_VOCAB = 1000000
_EMB = 64
_BATCH = 4096
_SEQ = 200
_CLS = 5

_NC = 2
_NS = 16
_NW = _NC * _NS
_BPW = _BATCH // _NW
_LANES = 16
_CHUNKS = _EMB // _LANES


def _pool_body(pairidx_hbm, off_hbm, table_hbm, out_hbm,
               idx0, idx1, off0, off1, rows_v, pooled_v,
               isem0, isem1, osem0, osem1, gsem0, gsem1):
    wid = lax.axis_index("s") * _NC + lax.axis_index("c")
    base = wid * _BPW
    fbase = base * _SEQ

    idx_bufs = (idx0, idx1)
    off_bufs = (off0, off1)
    isems = (isem0, isem1)
    osems = (osem0, osem1)
    gsems = (gsem0, gsem1)

    def idx_start(i, b):
        pltpu.async_copy(pairidx_hbm.at[pl.ds(fbase + i * _SEQ, _SEQ)],
                         idx_bufs[b], isems[b])
        pltpu.async_copy(off_hbm.at[pl.ds(fbase + i * _SEQ, _SEQ)],
                         off_bufs[b].at[pl.ds(0, _SEQ)], osems[b])

    def idx_wait(i, b):
        pltpu.make_async_copy(pairidx_hbm.at[pl.ds(fbase + i * _SEQ, _SEQ)],
                              idx_bufs[b], isems[b]).wait()
        pltpu.make_async_copy(off_hbm.at[pl.ds(fbase + i * _SEQ, _SEQ)],
                              off_bufs[b].at[pl.ds(0, _SEQ)], osems[b]).wait()

    def gather_start(b):
        pltpu.async_copy(table_hbm.at[idx_bufs[b]], rows_v.at[b], gsems[b])

    def gather_wait(b):
        pltpu.make_async_copy(table_hbm.at[idx_bufs[b]], rows_v.at[b],
                              gsems[b]).wait()

    def accumulate(i, buf):
        rows = rows_v.at[buf]
        offv = off_bufs[buf]
        zero = jnp.zeros((_LANES,), jnp.float32)

        def do_rows(t, k, ov, out):
            half = k % 2
            o = ov[k]
            for c in range(_CHUNKS):
                out[half * _CHUNKS + c] = (
                    out[half * _CHUNKS + c]
                    + rows[t + k, pl.ds(o + c * _LANES, _LANES)]
                )

        @plsc.parallel_loop(0, 192, 16,
                            carry=tuple(zero for _ in range(2 * _CHUNKS)))
        def acc(t, carry):
            out = list(carry)
            ov = offv[pl.ds(t, _LANES)]
            for k in range(_LANES):
                do_rows(t, k, ov, out)
            return tuple(out)

        out = list(acc)
        ov = offv[pl.ds(192, _LANES)]
        for k in range(_SEQ - 192):
            do_rows(192, k, ov, out)

        inv = jnp.float32(1.0 / _SEQ)
        for c in range(_CHUNKS):
            pooled_v[i, pl.ds(c * _LANES, _LANES)] = (
                (out[c] + out[_CHUNKS + c]) * inv
            )

    idx_start(0, 0)
    idx_wait(0, 0)
    gather_start(0)
    idx_start(1, 1)

    def step(i, b):
        gather_wait(b)

        @pl.when(i + 1 < _BPW)
        def _():
            idx_wait(i + 1, 1 - b)
            gather_start(1 - b)

        accumulate(i, b)

        @pl.when(i + 2 < _BPW)
        def _():
            idx_start(i + 2, b)

    def outer(g, _):
        step(g * 2, 0)
        step(g * 2 + 1, 1)
        return 0

    lax.fori_loop(0, _BPW // 2, outer, 0)

    pltpu.sync_copy(pooled_v, out_hbm.at[pl.ds(base, _BPW)])


_pool = pl.kernel(
    out_type=jax.ShapeDtypeStruct((_BATCH, _EMB), jnp.float32),
    mesh=plsc.VectorSubcoreMesh(core_axis_name="c", subcore_axis_name="s",
                                num_cores=_NC, num_subcores=_NS),
    scratch_types=[
        pltpu.VMEM((_SEQ,), jnp.int32),
        pltpu.VMEM((_SEQ,), jnp.int32),
        pltpu.VMEM((208,), jnp.int32),
        pltpu.VMEM((208,), jnp.int32),
        pltpu.VMEM((2, _SEQ, 2 * _EMB), jnp.float32),
        pltpu.VMEM((_BPW, _EMB), jnp.float32),
        pltpu.SemaphoreType.DMA,
        pltpu.SemaphoreType.DMA,
        pltpu.SemaphoreType.DMA,
        pltpu.SemaphoreType.DMA,
        pltpu.SemaphoreType.DMA,
        pltpu.SemaphoreType.DMA,
    ],
)(_pool_body)


def _linear_body(pooled_ref, wt_ref, b_ref, out_ref):
    out_ref[...] = (
        jnp.dot(pooled_ref[...], wt_ref[...],
                preferred_element_type=jnp.float32)
        + b_ref[...]
    )


def _linear(pooled, wt, b2):
    return pl.pallas_call(
        _linear_body,
        out_shape=jax.ShapeDtypeStruct((_BATCH, _CLS), jnp.float32),
    )(pooled, wt, b2)


def kernel(inputs, emb_table, W, b):
    pairidx = (inputs >> 1).reshape(-1)
    halfoff = ((inputs & 1) * _EMB).reshape(-1)
    table128 = emb_table.reshape(_VOCAB // 2, 2 * _EMB)
    pooled = _pool(pairidx, halfoff, table128)
    return _linear(pooled, W.T, b[None, :])


# speedup vs baseline: 1.5048x; 1.5048x over previous
import jax
import jax.numpy as jnp
from jax import lax
from jax.experimental import pallas as pl
from jax.experimental.pallas import tpu as pltpu
from jax.experimental.pallas import tpu_sc as plsc

_VOCAB = 1000000
_EMB = 64
_BATCH = 4096
_SEQ = 200
_CLS = 5

_NC = 2
_NS = 16
_NW = _NC * _NS
_BPW = _BATCH // _NW
_LANES = 16
_CHUNKS = _EMB // _LANES


def _pool_body(inputs_hbm, table_hbm, out_hbm, idx0, idx1, rows_v,
               pooled_v, isem0, isem1, gsem0, gsem1):
    wid = lax.axis_index("s") * _NC + lax.axis_index("c")
    base = wid * _BPW
    fbase = base * _SEQ

    idx_bufs = (idx0, idx1)
    isems = (isem0, isem1)
    gsems = (gsem0, gsem1)

    def idx_start(i, b):
        pltpu.async_copy(inputs_hbm.at[pl.ds(fbase + i * _SEQ, _SEQ)],
                         idx_bufs[b], isems[b])

    def idx_wait(i, b):
        pltpu.make_async_copy(inputs_hbm.at[pl.ds(fbase + i * _SEQ, _SEQ)],
                              idx_bufs[b], isems[b]).wait()

    def gather_start(b):
        pltpu.async_copy(table_hbm.at[idx_bufs[b]], rows_v.at[b], gsems[b])

    def gather_wait(b):
        pltpu.make_async_copy(table_hbm.at[idx_bufs[b]], rows_v.at[b],
                              gsems[b]).wait()

    def accumulate(i, buf):
        rows = rows_v.at[buf]
        zero = jnp.zeros((_LANES,), jnp.float32)

        @plsc.parallel_loop(0, _SEQ, 8,
                            carry=tuple(zero for _ in range(2 * _CHUNKS)))
        def acc(t, carry):
            out = list(carry)
            for k in range(8):
                half = k % 2
                for c in range(_CHUNKS):
                    out[half * _CHUNKS + c] = (
                        out[half * _CHUNKS + c]
                        + rows[t + k, pl.ds(c * _LANES, _LANES)]
                    )
            return tuple(out)

        inv = jnp.float32(1.0 / _SEQ)
        for c in range(_CHUNKS):
            pooled_v[i, pl.ds(c * _LANES, _LANES)] = (
                (acc[c] + acc[_CHUNKS + c]) * inv
            )

    idx_start(0, 0)
    idx_wait(0, 0)
    gather_start(0)
    idx_start(1, 1)

    def step(i, b):
        gather_wait(b)

        @pl.when(i + 2 < _BPW)
        def _():
            idx_start(i + 2, b)

        @pl.when(i + 1 < _BPW)
        def _():
            idx_wait(i + 1, 1 - b)
            gather_start(1 - b)

        accumulate(i, b)

    def outer(g, _):
        step(g * 2, 0)
        step(g * 2 + 1, 1)
        return 0

    lax.fori_loop(0, _BPW // 2, outer, 0)

    pltpu.sync_copy(pooled_v, out_hbm.at[pl.ds(base, _BPW)])


_pool = pl.kernel(
    out_type=jax.ShapeDtypeStruct((_BATCH, _EMB), jnp.float32),
    mesh=plsc.VectorSubcoreMesh(core_axis_name="c", subcore_axis_name="s",
                                num_cores=_NC, num_subcores=_NS),
    scratch_types=[
        pltpu.VMEM((_SEQ,), jnp.int32),
        pltpu.VMEM((_SEQ,), jnp.int32),
        pltpu.VMEM((2, _SEQ, _EMB), jnp.float32),
        pltpu.VMEM((_BPW, _EMB), jnp.float32),
        pltpu.SemaphoreType.DMA,
        pltpu.SemaphoreType.DMA,
        pltpu.SemaphoreType.DMA,
        pltpu.SemaphoreType.DMA,
    ],
    compiler_params=pltpu.CompilerParams(use_tc_tiling_on_sc=False),
)(_pool_body)


def _linear_body(pooled_ref, wt_ref, b_ref, out_ref):
    out_ref[...] = (
        jnp.dot(pooled_ref[...], wt_ref[...],
                preferred_element_type=jnp.float32)
        + b_ref[...]
    )


def _linear(pooled, wt, b2):
    return pl.pallas_call(
        _linear_body,
        out_shape=jax.ShapeDtypeStruct((_BATCH, _CLS), jnp.float32),
    )(pooled, wt, b2)


def kernel(inputs, emb_table, W, b):
    pooled = _pool(inputs.reshape(-1), emb_table)
    return _linear(pooled, W.T, b[None, :])


# speedup vs baseline: 1.6537x; 1.0990x over previous
import jax
import jax.numpy as jnp
from jax import lax
from jax.experimental import pallas as pl
from jax.experimental.pallas import tpu as pltpu
from jax.experimental.pallas import tpu_sc as plsc

_VOCAB = 1000000
_EMB = 64
_BATCH = 4096
_SEQ = 200
_CLS = 5

_NC = 2
_NS = 16
_NW = _NC * _NS
_BPW = _BATCH // _NW
_LANES = 16
_CHUNKS = _EMB // _LANES


_DEPTH = 4


def _pool_body(inputs_hbm, table_hbm, out_hbm, idx_v, rows_v, pooled_v,
               isem0, isem1, isem2, isem3, gsem0, gsem1, gsem2, gsem3):
    wid = lax.axis_index("s") * _NC + lax.axis_index("c")
    base = wid * _BPW
    fbase = base * _SEQ

    isems = (isem0, isem1, isem2, isem3)
    gsems = (gsem0, gsem1, gsem2, gsem3)

    def idx_start(i, b):
        pltpu.async_copy(inputs_hbm.at[pl.ds(fbase + i * _SEQ, _SEQ)],
                         idx_v.at[b], isems[b])

    def idx_wait(i, b):
        pltpu.make_async_copy(inputs_hbm.at[pl.ds(fbase + i * _SEQ, _SEQ)],
                              idx_v.at[b], isems[b]).wait()

    def gather_start(b):
        pltpu.async_copy(table_hbm.at[idx_v.at[b]], rows_v.at[b], gsems[b])

    def gather_wait(b):
        pltpu.make_async_copy(table_hbm.at[idx_v.at[b]], rows_v.at[b],
                              gsems[b]).wait()

    def accumulate(i, buf):
        rows = rows_v.at[buf]
        zero = jnp.zeros((_LANES,), jnp.float32)

        @plsc.parallel_loop(0, _SEQ, 8,
                            carry=tuple(zero for _ in range(2 * _CHUNKS)))
        def acc(t, carry):
            out = list(carry)
            for k in range(8):
                half = k % 2
                for c in range(_CHUNKS):
                    out[half * _CHUNKS + c] = (
                        out[half * _CHUNKS + c]
                        + rows[t + k, pl.ds(c * _LANES, _LANES)]
                    )
            return tuple(out)

        inv = jnp.float32(1.0 / _SEQ)
        for c in range(_CHUNKS):
            pooled_v[i, pl.ds(c * _LANES, _LANES)] = (
                (acc[c] + acc[_CHUNKS + c]) * inv
            )

    for d in range(_DEPTH):
        idx_start(d, d)
    for d in range(_DEPTH):
        idx_wait(d, d)
        gather_start(d)

    def step(i, b):
        gather_wait(b)

        @pl.when(i + _DEPTH < _BPW)
        def _():
            idx_start(i + _DEPTH, b)

        accumulate(i, b)

        @pl.when(i + _DEPTH < _BPW)
        def _():
            idx_wait(i + _DEPTH, b)
            gather_start(b)

    def outer(g, _):
        for d in range(_DEPTH):
            step(g * _DEPTH + d, d)
        return 0

    lax.fori_loop(0, _BPW // _DEPTH, outer, 0)

    pltpu.sync_copy(pooled_v, out_hbm.at[pl.ds(base, _BPW)])


_pool = pl.kernel(
    out_type=jax.ShapeDtypeStruct((_BATCH, _EMB), jnp.float32),
    mesh=plsc.VectorSubcoreMesh(core_axis_name="c", subcore_axis_name="s",
                                num_cores=_NC, num_subcores=_NS),
    scratch_types=[
        pltpu.VMEM((_DEPTH, _SEQ), jnp.int32),
        pltpu.VMEM((_DEPTH, _SEQ, _EMB), jnp.float32),
        pltpu.VMEM((_BPW, _EMB), jnp.float32),
        pltpu.SemaphoreType.DMA,
        pltpu.SemaphoreType.DMA,
        pltpu.SemaphoreType.DMA,
        pltpu.SemaphoreType.DMA,
        pltpu.SemaphoreType.DMA,
        pltpu.SemaphoreType.DMA,
        pltpu.SemaphoreType.DMA,
        pltpu.SemaphoreType.DMA,
    ],
    compiler_params=pltpu.CompilerParams(use_tc_tiling_on_sc=False),
)(_pool_body)


def _linear_body(pooled_ref, wt_ref, b_ref, out_ref):
    out_ref[...] = (
        jnp.dot(pooled_ref[...], wt_ref[...],
                preferred_element_type=jnp.float32)
        + b_ref[...]
    )


def _linear(pooled, wt, b2):
    return pl.pallas_call(
        _linear_body,
        out_shape=jax.ShapeDtypeStruct((_BATCH, _CLS), jnp.float32),
    )(pooled, wt, b2)


def kernel(inputs, emb_table, W, b):
    pooled = _pool(inputs.reshape(-1), emb_table)
    return _linear(pooled, W.T, b[None, :])


# speedup vs baseline: 1.6836x; 1.0180x over previous
import jax
import jax.numpy as jnp
from jax import lax
from jax.experimental import pallas as pl
from jax.experimental.pallas import tpu as pltpu
from jax.experimental.pallas import tpu_sc as plsc

_VOCAB = 1000000
_EMB = 64
_BATCH = 4096
_SEQ = 200
_CLS = 5

_NC = 2
_NS = 16
_NW = _NC * _NS
_BPW = _BATCH // _NW
_LANES = 16
_CHUNKS = _EMB // _LANES


_DEPTH = 4


def _pool_body(inputs_hbm, table_hbm, out_hbm, idx_v, rows_v, pooled_v,
               isem, gsem0, gsem1, gsem2, gsem3):
    wid = lax.axis_index("s") * _NC + lax.axis_index("c")
    base = wid * _BPW
    fbase = base * _SEQ

    gsems = (gsem0, gsem1, gsem2, gsem3)

    def idx_slice(i):
        return idx_v.at[pl.ds(i * _SEQ, _SEQ)]

    def gather_start(i, b):
        pltpu.async_copy(table_hbm.at[idx_slice(i)], rows_v.at[b], gsems[b])

    def gather_wait(i, b):
        pltpu.make_async_copy(table_hbm.at[idx_slice(i)], rows_v.at[b],
                              gsems[b]).wait()

    def accumulate(i, buf):
        rows = rows_v.at[buf]
        zero = jnp.zeros((_LANES,), jnp.float32)

        @plsc.parallel_loop(0, _SEQ, 8,
                            carry=tuple(zero for _ in range(2 * _CHUNKS)))
        def acc(t, carry):
            out = list(carry)
            for k in range(8):
                half = k % 2
                for c in range(_CHUNKS):
                    out[half * _CHUNKS + c] = (
                        out[half * _CHUNKS + c]
                        + rows[t + k, pl.ds(c * _LANES, _LANES)]
                    )
            return tuple(out)

        inv = jnp.float32(1.0 / _SEQ)
        for c in range(_CHUNKS):
            pooled_v[i, pl.ds(c * _LANES, _LANES)] = (
                (acc[c] + acc[_CHUNKS + c]) * inv
            )

    pltpu.async_copy(inputs_hbm.at[pl.ds(fbase, _BPW * _SEQ)], idx_v, isem)
    pltpu.make_async_copy(inputs_hbm.at[pl.ds(fbase, _BPW * _SEQ)],
                          idx_v, isem).wait()

    for d in range(_DEPTH):
        gather_start(d, d)

    def step(i, b):
        gather_wait(i, b)
        accumulate(i, b)

        @pl.when(i + _DEPTH < _BPW)
        def _():
            gather_start(i + _DEPTH, b)

    def outer(g, _):
        for d in range(_DEPTH):
            step(g * _DEPTH + d, d)
        return 0

    lax.fori_loop(0, _BPW // _DEPTH, outer, 0)

    pltpu.sync_copy(pooled_v, out_hbm.at[pl.ds(base, _BPW)])


_pool = pl.kernel(
    out_type=jax.ShapeDtypeStruct((_BATCH, _EMB), jnp.float32),
    mesh=plsc.VectorSubcoreMesh(core_axis_name="c", subcore_axis_name="s",
                                num_cores=_NC, num_subcores=_NS),
    scratch_types=[
        pltpu.VMEM((_BPW * _SEQ,), jnp.int32),
        pltpu.VMEM((_DEPTH, _SEQ, _EMB), jnp.float32),
        pltpu.VMEM((_BPW, _EMB), jnp.float32),
        pltpu.SemaphoreType.DMA,
        pltpu.SemaphoreType.DMA,
        pltpu.SemaphoreType.DMA,
        pltpu.SemaphoreType.DMA,
        pltpu.SemaphoreType.DMA,
    ],
    compiler_params=pltpu.CompilerParams(use_tc_tiling_on_sc=False),
)(_pool_body)


def _linear_body(pooled_ref, wt_ref, b_ref, out_ref):
    out_ref[...] = (
        jnp.dot(pooled_ref[...], wt_ref[...],
                preferred_element_type=jnp.float32)
        + b_ref[...]
    )


def _linear(pooled, wt, b2):
    return pl.pallas_call(
        _linear_body,
        out_shape=jax.ShapeDtypeStruct((_BATCH, _CLS), jnp.float32),
    )(pooled, wt, b2)


def kernel(inputs, emb_table, W, b):
    pooled = _pool(inputs.reshape(-1), emb_table)
    return _linear(pooled, W.T, b[None, :])
